# baseline (speedup 1.0000x reference)
import functools

import jax
import jax.numpy as jnp
from jax import lax
from jax.experimental import pallas as pl
from jax.experimental.pallas import tpu as pltpu
from jax.experimental.pallas import tpu_sc as plsc

R = 128
C = 32768
NC = 2
NS = 16
NW = NC * NS
RPW = R // NW
NV = C // 16
NACC = 4
NGRP = 16
VPI = NACC * NGRP
NIT = NV // VPI
NBUF = 3

_mesh = plsc.VectorSubcoreMesh(core_axis_name="c", subcore_axis_name="s")


@functools.partial(
    pl.kernel,
    out_type=jax.ShapeDtypeStruct((NW, 16), jnp.int32),
    mesh=_mesh,
    compiler_params=pltpu.CompilerParams(needs_layout_passes=False),
    scratch_types=[
        pltpu.VMEM((C,), jnp.float32),
        pltpu.VMEM((C,), jnp.float32),
        pltpu.VMEM((C,), jnp.float32),
        pltpu.VMEM((16,), jnp.int32),
        pltpu.SemaphoreType.DMA,
        pltpu.SemaphoreType.DMA,
        pltpu.SemaphoreType.DMA,
    ],
)
def _argmax_sc(x_hbm, out_hbm, buf0, buf1, buf2, res_v, sem0, sem1, sem2):
    wid = lax.axis_index("s") * NC + lax.axis_index("c")
    lane = lax.iota(jnp.int32, 16)
    bufs = (buf0, buf1, buf2)
    sems = (sem0, sem1, sem2)
    row0 = wid * RPW

    for rl in range(NBUF - 1):
        pltpu.make_async_copy(
            x_hbm.at[row0 + rl], bufs[rl], sems[rl]).start()

    res_vec = jnp.zeros((16,), jnp.int32)
    for rl in range(RPW):
        b = bufs[rl % NBUF]
        pltpu.make_async_copy(
            x_hbm.at[row0 + rl], b, sems[rl % NBUF]).wait()
        nxt = rl + NBUF - 1
        if nxt < RPW:
            pltpu.make_async_copy(
                x_hbm.at[row0 + nxt],
                bufs[nxt % NBUF], sems[nxt % NBUF]).start()

        neg_inf = jnp.full((16,), -jnp.inf, jnp.float32)
        zero = jnp.zeros((16,), jnp.int32)
        init = (neg_inf,) * NACC + (zero,) * NACC

        def body(i, carry, b=b):
            cmax = list(carry[:NACC])
            crec = list(carry[NACC:])
            base = i * VPI
            for g in range(NGRP):
                for k in range(NACC):
                    gi = base + g * NACC + k
                    val = b[pl.ds(gi * 16, 16)]
                    m = val > cmax[k]
                    cmax[k] = jnp.where(m, val, cmax[k])
                    crec[k] = jnp.where(m, gi, crec[k])
            return tuple(cmax) + tuple(crec)

        acc = lax.fori_loop(0, NIT, body, init)
        cmax = list(acc[:NACC])
        crec = list(acc[NACC:])

        n = NACC
        while n > 1:
            for k in range(n // 2):
                av, bv = cmax[2 * k], cmax[2 * k + 1]
                ar, br = crec[2 * k], crec[2 * k + 1]
                take_a = (av > bv) | ((av == bv) & (ar < br))
                cmax[k] = jnp.where(take_a, av, bv)
                crec[k] = jnp.where(take_a, ar, br)
            n //= 2

        m = jnp.max(cmax[0])
        idx = crec[0] * 16 + lane
        cand = jnp.where(cmax[0] == m, idx, jnp.int32(0x7FFFFFFF))
        best = jnp.min(cand)
        res_vec = jnp.where(lane == rl, best, res_vec)

    res_v[...] = res_vec
    pltpu.sync_copy(res_v, out_hbm.at[wid])


def kernel(input_data):
    out = _argmax_sc(input_data)
    return out[:, :RPW].reshape(R, 1).astype(jnp.int64)


# speedup vs baseline: 1.0373x; 1.0373x over previous
import functools

import jax
import jax.numpy as jnp
from jax import lax
from jax.experimental import pallas as pl
from jax.experimental.pallas import tpu as pltpu
from jax.experimental.pallas import tpu_sc as plsc

R = 128
C = 32768
NC = 2
NS = 16
NW = NC * NS
RPW = R // NW
NV = C // 16
NACC = 4
NGRP = 16
VPI = NACC * NGRP
NIT = NV // VPI
NBUF = 2

_mesh = plsc.VectorSubcoreMesh(core_axis_name="c", subcore_axis_name="s")


@functools.partial(
    pl.kernel,
    out_type=jax.ShapeDtypeStruct((NW, 16), jnp.int32),
    mesh=_mesh,
    compiler_params=pltpu.CompilerParams(needs_layout_passes=False),
    scratch_types=[
        pltpu.VMEM((C,), jnp.float32),
        pltpu.VMEM((C,), jnp.float32),
        pltpu.VMEM((16,), jnp.int32),
        pltpu.SemaphoreType.DMA,
        pltpu.SemaphoreType.DMA,
    ],
)
def _argmax_sc(x_hbm, out_hbm, buf0, buf1, res_v, sem0, sem1):
    wid = lax.axis_index("s") * NC + lax.axis_index("c")
    lane = lax.iota(jnp.int32, 16)
    bufs = (buf0, buf1)
    sems = (sem0, sem1)
    row0 = wid * RPW

    for rl in range(NBUF - 1):
        pltpu.make_async_copy(
            x_hbm.at[row0 + rl], bufs[rl], sems[rl]).start()

    res_vec = jnp.zeros((16,), jnp.int32)
    for rl in range(RPW):
        b = bufs[rl % NBUF]
        pltpu.make_async_copy(
            x_hbm.at[row0 + rl], b, sems[rl % NBUF]).wait()
        nxt = rl + NBUF - 1
        if nxt < RPW:
            pltpu.make_async_copy(
                x_hbm.at[row0 + nxt],
                bufs[nxt % NBUF], sems[nxt % NBUF]).start()

        neg_inf = jnp.full((16,), -jnp.inf, jnp.float32)
        zero = jnp.zeros((16,), jnp.int32)
        init = (neg_inf,) * NACC + (zero,) * NACC

        def body(i, carry, b=b):
            cmax = list(carry[:NACC])
            crec = list(carry[NACC:])
            base = i * VPI
            for g in range(NGRP):
                for k in range(NACC):
                    gi = base + g * NACC + k
                    val = b[pl.ds(gi * 16, 16)]
                    m = val > cmax[k]
                    cmax[k] = jnp.where(m, val, cmax[k])
                    crec[k] = jnp.where(m, gi, crec[k])
            return tuple(cmax) + tuple(crec)

        acc = lax.fori_loop(0, NIT, body, init)
        cmax = list(acc[:NACC])
        crec = list(acc[NACC:])

        n = NACC
        while n > 1:
            for k in range(n // 2):
                av, bv = cmax[2 * k], cmax[2 * k + 1]
                ar, br = crec[2 * k], crec[2 * k + 1]
                take_a = (av > bv) | ((av == bv) & (ar < br))
                cmax[k] = jnp.where(take_a, av, bv)
                crec[k] = jnp.where(take_a, ar, br)
            n //= 2

        m = jnp.max(cmax[0])
        idx = crec[0] * 16 + lane
        cand = jnp.where(cmax[0] == m, idx, jnp.int32(0x7FFFFFFF))
        best = jnp.min(cand)
        res_vec = jnp.where(lane == rl, best, res_vec)

    res_v[...] = res_vec
    pltpu.sync_copy(res_v, out_hbm.at[wid])


def kernel(input_data):
    out = _argmax_sc(input_data)
    return out[:, :RPW].reshape(R, 1).astype(jnp.int64)


# speedup vs baseline: 1.2052x; 1.1618x over previous
import functools

import jax
import jax.numpy as jnp
from jax import lax
from jax.experimental import pallas as pl
from jax.experimental.pallas import tpu as pltpu
from jax.experimental.pallas import tpu_sc as plsc

R = 128
C = 32768
NC = 2
NS = 16
NW = NC * NS
RPW = R // NW
NV = C // 16
NACC = 4
NGRP = 4
VPI = NACC * NGRP
NIT = NV // VPI
NBUF = 2

_mesh = plsc.VectorSubcoreMesh(core_axis_name="c", subcore_axis_name="s")


@functools.partial(
    pl.kernel,
    out_type=jax.ShapeDtypeStruct((NW, 16), jnp.int32),
    mesh=_mesh,
    compiler_params=pltpu.CompilerParams(needs_layout_passes=False),
    scratch_types=[
        pltpu.VMEM((C,), jnp.float32),
        pltpu.VMEM((C,), jnp.float32),
        pltpu.VMEM((16,), jnp.int32),
        pltpu.SemaphoreType.DMA,
        pltpu.SemaphoreType.DMA,
    ],
)
def _argmax_sc(x_hbm, out_hbm, buf0, buf1, res_v, sem0, sem1):
    wid = lax.axis_index("s") * NC + lax.axis_index("c")
    lane = lax.iota(jnp.int32, 16)
    bufs = (buf0, buf1)
    sems = (sem0, sem1)
    row0 = wid * RPW

    for rl in range(NBUF - 1):
        pltpu.make_async_copy(
            x_hbm.at[row0 + rl], bufs[rl], sems[rl]).start()

    res_vec = jnp.zeros((16,), jnp.int32)
    for rl in range(RPW):
        b = bufs[rl % NBUF]
        pltpu.make_async_copy(
            x_hbm.at[row0 + rl], b, sems[rl % NBUF]).wait()
        nxt = rl + NBUF - 1
        if nxt < RPW:
            pltpu.make_async_copy(
                x_hbm.at[row0 + nxt],
                bufs[nxt % NBUF], sems[nxt % NBUF]).start()

        neg_inf = jnp.full((16,), -jnp.inf, jnp.float32)
        zero = jnp.zeros((16,), jnp.int32)
        init = (neg_inf,) * NACC + (zero,) * NACC

        def body(i, carry, b=b):
            cmax = list(carry[:NACC])
            crec = list(carry[NACC:])
            base = i * VPI
            for g in range(NGRP):
                for k in range(NACC):
                    gi = base + g * NACC + k
                    val = b[pl.ds(gi * 16, 16)]
                    m = val > cmax[k]
                    cmax[k] = jnp.where(m, val, cmax[k])
                    crec[k] = jnp.where(m, gi, crec[k])
            return tuple(cmax) + tuple(crec)

        acc = lax.fori_loop(0, NIT, body, init)
        cmax = list(acc[:NACC])
        crec = list(acc[NACC:])

        n = NACC
        while n > 1:
            for k in range(n // 2):
                av, bv = cmax[2 * k], cmax[2 * k + 1]
                ar, br = crec[2 * k], crec[2 * k + 1]
                take_a = (av > bv) | ((av == bv) & (ar < br))
                cmax[k] = jnp.where(take_a, av, bv)
                crec[k] = jnp.where(take_a, ar, br)
            n //= 2

        m = jnp.max(cmax[0])
        idx = crec[0] * 16 + lane
        cand = jnp.where(cmax[0] == m, idx, jnp.int32(0x7FFFFFFF))
        best = jnp.min(cand)
        res_vec = jnp.where(lane == rl, best, res_vec)

    res_v[...] = res_vec
    pltpu.sync_copy(res_v, out_hbm.at[wid])


def kernel(input_data):
    out = _argmax_sc(input_data)
    return out[:, :RPW].reshape(R, 1).astype(jnp.int64)


# speedup vs baseline: 1.2232x; 1.0149x over previous
import functools

import jax
import jax.numpy as jnp
from jax import lax
from jax.experimental import pallas as pl
from jax.experimental.pallas import tpu as pltpu
from jax.experimental.pallas import tpu_sc as plsc

R = 128
C = 32768
NC = 2
NS = 16
NW = NC * NS
RPW = R // NW
NV = C // 16
NACC = 4
NGRP = 2
VPI = NACC * NGRP
NIT = NV // VPI
NBUF = 2

_mesh = plsc.VectorSubcoreMesh(core_axis_name="c", subcore_axis_name="s")


@functools.partial(
    pl.kernel,
    out_type=jax.ShapeDtypeStruct((NW, 16), jnp.int32),
    mesh=_mesh,
    compiler_params=pltpu.CompilerParams(needs_layout_passes=False),
    scratch_types=[
        pltpu.VMEM((C,), jnp.float32),
        pltpu.VMEM((C,), jnp.float32),
        pltpu.VMEM((16,), jnp.int32),
        pltpu.SemaphoreType.DMA,
        pltpu.SemaphoreType.DMA,
    ],
)
def _argmax_sc(x_hbm, out_hbm, buf0, buf1, res_v, sem0, sem1):
    wid = lax.axis_index("s") * NC + lax.axis_index("c")
    lane = lax.iota(jnp.int32, 16)
    bufs = (buf0, buf1)
    sems = (sem0, sem1)
    row0 = wid * RPW

    for rl in range(NBUF - 1):
        pltpu.make_async_copy(
            x_hbm.at[row0 + rl], bufs[rl], sems[rl]).start()

    res_vec = jnp.zeros((16,), jnp.int32)
    for rl in range(RPW):
        b = bufs[rl % NBUF]
        pltpu.make_async_copy(
            x_hbm.at[row0 + rl], b, sems[rl % NBUF]).wait()
        nxt = rl + NBUF - 1
        if nxt < RPW:
            pltpu.make_async_copy(
                x_hbm.at[row0 + nxt],
                bufs[nxt % NBUF], sems[nxt % NBUF]).start()

        neg_inf = jnp.full((16,), -jnp.inf, jnp.float32)
        zero = jnp.zeros((16,), jnp.int32)
        init = (neg_inf,) * NACC + (zero,) * NACC

        def body(i, carry, b=b):
            cmax = list(carry[:NACC])
            crec = list(carry[NACC:])
            base = i * VPI
            for g in range(NGRP):
                for k in range(NACC):
                    gi = base + g * NACC + k
                    val = b[pl.ds(gi * 16, 16)]
                    m = val > cmax[k]
                    cmax[k] = jnp.where(m, val, cmax[k])
                    crec[k] = jnp.where(m, gi, crec[k])
            return tuple(cmax) + tuple(crec)

        acc = lax.fori_loop(0, NIT, body, init)
        cmax = list(acc[:NACC])
        crec = list(acc[NACC:])

        n = NACC
        while n > 1:
            for k in range(n // 2):
                av, bv = cmax[2 * k], cmax[2 * k + 1]
                ar, br = crec[2 * k], crec[2 * k + 1]
                take_a = (av > bv) | ((av == bv) & (ar < br))
                cmax[k] = jnp.where(take_a, av, bv)
                crec[k] = jnp.where(take_a, ar, br)
            n //= 2

        m = jnp.max(cmax[0])
        idx = crec[0] * 16 + lane
        cand = jnp.where(cmax[0] == m, idx, jnp.int32(0x7FFFFFFF))
        best = jnp.min(cand)
        res_vec = jnp.where(lane == rl, best, res_vec)

    res_v[...] = res_vec
    pltpu.sync_copy(res_v, out_hbm.at[wid])


def kernel(input_data):
    out = _argmax_sc(input_data)
    return out[:, :RPW].reshape(R, 1).astype(jnp.int64)
